# baseline (speedup 1.0000x reference)
import jax
import jax.numpy as jnp
from jax import lax
from jax.experimental import pallas as pl
from jax.experimental.pallas import tpu as pltpu
from jax.experimental.pallas import tpu_sc as plsc

_BM = 512

_NC, _NS = 2, 16
_NW = _NC * _NS
_SC_ROWS = 1024
_CHUNK = 8


def _tc_body(x_ref, adj_ref, w_ref, o_ref, xw_ref):
    @pl.when(pl.program_id(0) == 0)
    def _():
        xw = jax.lax.dot_general(
            x_ref[...], w_ref[...], (((1,), (1,)), ((), ())),
            preferred_element_type=jnp.float32)
        xw_ref[...] = xw.astype(jnp.bfloat16)

    adjb = adj_ref[...].astype(jnp.bfloat16)
    y = jax.lax.dot_general(
        adjb, xw_ref[...], (((1,), (0,)), ((), ())),
        preferred_element_type=jnp.float32)
    o_ref[...] = jnp.maximum(y, 0.0)


def _tc_call(x, adj, W):
    n, d_in = x.shape
    d_out = W.shape[0]
    return pl.pallas_call(
        _tc_body,
        grid=(n // _BM,),
        in_specs=[
            pl.BlockSpec((n, d_in), lambda i: (0, 0)),
            pl.BlockSpec((_BM, n), lambda i: (i, 0)),
            pl.BlockSpec((d_out, d_in), lambda i: (0, 0)),
        ],
        out_specs=pl.BlockSpec((_BM, d_out), lambda i: (i, 0)),
        out_shape=jax.ShapeDtypeStruct((n, d_out), jnp.float32),
        scratch_shapes=[pltpu.VMEM((n, d_out), jnp.bfloat16)],
        cost_estimate=pl.CostEstimate(
            flops=2 * n * n * d_out + 2 * n * d_in * d_out,
            bytes_accessed=(n * n + 2 * n * d_in + d_in * d_out) * 4,
            transcendentals=0,
        ),
    )(x, adj, W)


def _sc_probe(adj):
    n = adj.shape[0]
    r_per = _SC_ROWS // _NW
    nchunks = r_per // _CHUNK
    mesh = plsc.VectorSubcoreMesh(core_axis_name="c", subcore_axis_name="s")

    def body(adj_hbm, out_hbm, buf0, buf1, acc, sem):
        wid = lax.axis_index("s") * _NC + lax.axis_index("c")
        base = wid * r_per

        def issue(i, carry):
            @pl.when(i % 2 == 0)
            def _():
                pltpu.make_async_copy(
                    adj_hbm.at[pl.ds(base + i * _CHUNK, _CHUNK)], buf0, sem
                ).start()

            @pl.when(i % 2 == 1)
            def _():
                pltpu.make_async_copy(
                    adj_hbm.at[pl.ds(base + i * _CHUNK, _CHUNK)], buf1, sem
                ).start()

            return carry

        lax.fori_loop(0, nchunks, issue, 0)

        def drain(i, carry):
            pltpu.make_async_copy(
                adj_hbm.at[pl.ds(base, _CHUNK)], buf0, sem
            ).wait()
            return carry

        lax.fori_loop(0, nchunks, drain, 0)
        acc[...] = buf0[0, 0:16] + buf1[0, 0:16]
        pltpu.sync_copy(acc, out_hbm.at[wid])

    f = pl.kernel(
        body,
        out_type=jax.ShapeDtypeStruct((_NW, 16), jnp.float32),
        mesh=mesh,
        scratch_types=[
            pltpu.VMEM((_CHUNK, n), jnp.float32),
            pltpu.VMEM((_CHUNK, n), jnp.float32),
            pltpu.VMEM((16,), jnp.float32),
            pltpu.SemaphoreType.DMA,
        ],
        cost_estimate=pl.CostEstimate(
            flops=0,
            bytes_accessed=_SC_ROWS * n * 4,
            transcendentals=0,
        ),
    )
    return f(adj)


def kernel(x, adj, W):
    y = jax.nn.relu(adj @ (x @ W.T))
    s = _sc_probe(adj)
    return y.at[0, 0].add(0.0 * s[0, 0])


# speedup vs baseline: 1.7929x; 1.7929x over previous
import jax
import jax.numpy as jnp
from jax.experimental import pallas as pl
from jax.experimental.pallas import tpu as pltpu

_BM = 256


def _body(x_ref, adj_ref, w_ref, o_ref, xw_ref):
    @pl.when(pl.program_id(0) == 0)
    def _():
        xw = jax.lax.dot_general(
            x_ref[...], w_ref[...], (((1,), (1,)), ((), ())),
            preferred_element_type=jnp.float32)
        xw_ref[...] = xw.astype(jnp.bfloat16)

    adjb = adj_ref[...].astype(jnp.bfloat16)
    y = jax.lax.dot_general(
        adjb, xw_ref[...], (((1,), (0,)), ((), ())),
        preferred_element_type=jnp.float32)
    o_ref[...] = jnp.maximum(y, 0.0)


def kernel(x, adj, W):
    n, d_in = x.shape
    d_out = W.shape[0]
    return pl.pallas_call(
        _body,
        grid=(n // _BM,),
        in_specs=[
            pl.BlockSpec((n, d_in), lambda i: (0, 0)),
            pl.BlockSpec((_BM, n), lambda i: (i, 0)),
            pl.BlockSpec((d_out, d_in), lambda i: (0, 0)),
        ],
        out_specs=pl.BlockSpec((_BM, d_out), lambda i: (i, 0)),
        out_shape=jax.ShapeDtypeStruct((n, d_out), jnp.float32),
        scratch_shapes=[pltpu.VMEM((n, d_out), jnp.bfloat16)],
    )(x, adj, W)


# speedup vs baseline: 1.9638x; 1.0953x over previous
import jax
import jax.numpy as jnp
from jax.experimental import pallas as pl
from jax.experimental.pallas import tpu as pltpu

_BM = 1024


def _body(x_ref, adj_ref, w_ref, o_ref, xw_ref):
    @pl.when(pl.program_id(0) == 0)
    def _():
        xw = jax.lax.dot_general(
            x_ref[...], w_ref[...], (((1,), (1,)), ((), ())),
            preferred_element_type=jnp.float32)
        xw_ref[...] = xw.astype(jnp.bfloat16)

    adjb = adj_ref[...].astype(jnp.bfloat16)
    y = jax.lax.dot_general(
        adjb, xw_ref[...], (((1,), (0,)), ((), ())),
        preferred_element_type=jnp.float32)
    o_ref[...] = jnp.maximum(y, 0.0)


def kernel(x, adj, W):
    n, d_in = x.shape
    d_out = W.shape[0]
    return pl.pallas_call(
        _body,
        grid=(n // _BM,),
        in_specs=[
            pl.BlockSpec((n, d_in), lambda i: (0, 0)),
            pl.BlockSpec((_BM, n), lambda i: (i, 0)),
            pl.BlockSpec((d_out, d_in), lambda i: (0, 0)),
        ],
        out_specs=pl.BlockSpec((_BM, d_out), lambda i: (i, 0)),
        out_shape=jax.ShapeDtypeStruct((n, d_out), jnp.float32),
        scratch_shapes=[pltpu.VMEM((n, d_out), jnp.bfloat16)],
    )(x, adj, W)
